# Initial kernel scaffold; baseline (speedup 1.0000x reference)
import jax
import jax.numpy as jnp
from jax.experimental import pallas as pl


def kernel(n_feat, edge_index, e_feat, W1, al1, ar1, b1, W2, al2, ar2, b2):
    raise NotImplementedError("write your pallas kernel here")


# speedup vs baseline: 57.3682x; 57.3682x over previous
import functools

import jax
import jax.numpy as jnp
from jax import lax
from jax.experimental import pallas as pl
from jax.experimental.pallas import tpu as pltpu
from jax.experimental.pallas import tpu_sc as plsc

N = 10000
D = 128
H = 8
F = 16
TW = D + 2 * H
N_CORES = 2
N_SUB = 16
N_WORK = N_CORES * N_SUB
CH = 128
E_RAW = 320000
CHUNKS = -(-E_RAW // (N_WORK * CH))
E_PAD = N_WORK * CH * CHUNKS
EPW = CH * CHUNKS
N_ACC = 10112
ROWS_PER_TILE = N_ACC // N_SUB


def _embed_body(x_ref, w_ref, alr_ref, arr_ref, t_ref, er_ref):
    h = jnp.dot(x_ref[...], w_ref[...], preferred_element_type=jnp.float32)
    t_ref[:, :D] = h
    t_ref[:, D:TW] = jnp.dot(h, alr_ref[...], preferred_element_type=jnp.float32)
    er_ref[...] = jnp.dot(h, arr_ref[...], preferred_element_type=jnp.float32)


def _embed(x, w, alr, arr, rows_blk):
    n = x.shape[0]
    return pl.pallas_call(
        _embed_body,
        grid=(n // rows_blk,),
        in_specs=[
            pl.BlockSpec((rows_blk, D), lambda i: (i, 0)),
            pl.BlockSpec((D, D), lambda i: (0, 0)),
            pl.BlockSpec((D, 2 * H), lambda i: (0, 0)),
            pl.BlockSpec((D, 2 * H), lambda i: (0, 0)),
        ],
        out_specs=[
            pl.BlockSpec((rows_blk, TW), lambda i: (i, 0)),
            pl.BlockSpec((rows_blk, 2 * H), lambda i: (i, 0)),
        ],
        out_shape=[
            jax.ShapeDtypeStruct((n, TW), jnp.float32),
            jax.ShapeDtypeStruct((n, 2 * H), jnp.float32),
        ],
    )(x, w, alr, arr)


def _norm_embed_body(acc_ref, rep_ref, b_ref, w_ref, alr_ref, arr_ref,
                     t_ref, er_ref):
    s = acc_ref[0] + acc_ref[1]
    den = jnp.dot(s[:, D:D + H], rep_ref[...], preferred_element_type=jnp.float32)
    x1 = s[:, :D] / (den + 1e-9) + b_ref[...]
    x1 = jnp.maximum(x1, 0.01 * x1)
    h = jnp.dot(x1, w_ref[...], preferred_element_type=jnp.float32)
    t_ref[:, :D] = h
    t_ref[:, D:TW] = jnp.dot(h, alr_ref[...], preferred_element_type=jnp.float32)
    er_ref[...] = jnp.dot(h, arr_ref[...], preferred_element_type=jnp.float32)


def _norm_embed(acc, rep, b, w, alr, arr, rows_blk):
    n = acc.shape[1]
    return pl.pallas_call(
        _norm_embed_body,
        grid=(n // rows_blk,),
        in_specs=[
            pl.BlockSpec((2, rows_blk, TW), lambda i: (0, i, 0)),
            pl.BlockSpec((H, D), lambda i: (0, 0)),
            pl.BlockSpec((1, D), lambda i: (0, 0)),
            pl.BlockSpec((D, D), lambda i: (0, 0)),
            pl.BlockSpec((D, 2 * H), lambda i: (0, 0)),
            pl.BlockSpec((D, 2 * H), lambda i: (0, 0)),
        ],
        out_specs=[
            pl.BlockSpec((rows_blk, TW), lambda i: (i, 0)),
            pl.BlockSpec((rows_blk, 2 * H), lambda i: (i, 0)),
        ],
        out_shape=[
            jax.ShapeDtypeStruct((n, TW), jnp.float32),
            jax.ShapeDtypeStruct((n, 2 * H), jnp.float32),
        ],
    )(acc, rep, b, w, alr, arr)


def _final_body(acc_ref, rep_ref, b_ref, o_ref):
    s = acc_ref[0] + acc_ref[1]
    den = jnp.dot(s[:, D:D + H], rep_ref[...], preferred_element_type=jnp.float32)
    o_ref[...] = s[:, :D] / (den + 1e-9) + b_ref[...]


def _final(acc, rep, b, rows_blk):
    return pl.pallas_call(
        _final_body,
        grid=(N // rows_blk,),
        in_specs=[
            pl.BlockSpec((2, rows_blk, TW), lambda i: (0, i, 0)),
            pl.BlockSpec((H, D), lambda i: (0, 0)),
            pl.BlockSpec((1, D), lambda i: (0, 0)),
        ],
        out_specs=pl.BlockSpec((rows_blk, D), lambda i: (i, 0)),
        out_shape=jax.ShapeDtypeStruct((N, D), jnp.float32),
    )(acc, rep, b)


@functools.cache
def _make_sc_edge():
    mesh = plsc.VectorSubcoreMesh(core_axis_name="c", subcore_axis_name="s")
    return functools.partial(
        pl.kernel,
        mesh=mesh,
        compiler_params=pltpu.CompilerParams(use_tc_tiling_on_sc=False),
        out_type=jax.ShapeDtypeStruct((N_CORES, N_ACC, TW), jnp.float32),
        scratch_types=[
            pltpu.VMEM((CH,), jnp.int32),
            pltpu.VMEM((CH,), jnp.int32),
            pltpu.VMEM((CH, TW), jnp.float32),
            pltpu.VMEM((CH, 2 * H), jnp.float32),
            pltpu.VMEM_SHARED((N_ACC, TW), jnp.float32),
            pltpu.SemaphoreType.DMA,
            pltpu.SemaphoreType.DMA,
        ],
    )(_sc_edge_body)


def _sc_edge_body(t_hbm, er_hbm, src_hbm, dst_hbm, zero_hbm, out_hbm,
                  idx_s, idx_d, rows, errs, acc, sem_a, sem_b):
    c = lax.axis_index("c")
    s = lax.axis_index("s")
    pltpu.sync_copy(zero_hbm, acc.at[pl.ds(s * ROWS_PER_TILE, ROWS_PER_TILE)])
    plsc.subcore_barrier()

    base0 = c * (EPW * N_SUB) + s * EPW

    def chunk_body(i, carry):
        base = base0 + i * CH
        pltpu.sync_copy(src_hbm.at[pl.ds(base, CH)], idx_s)
        pltpu.sync_copy(dst_hbm.at[pl.ds(base, CH)], idx_d)
        ga = pltpu.async_copy(t_hbm.at[idx_s], rows, sem_a)
        gb = pltpu.async_copy(er_hbm.at[idx_d], errs, sem_b)
        ga.wait()
        gb.wait()

        def edge_body(e, carry2):
            ev = rows[e, pl.ds(D, 16)] + errs[e, :]
            ev = jnp.maximum(ev, 0.2 * ev)
            wv = jnp.exp(ev)
            rows[e, pl.ds(D, 16)] = wv
            for hh in range(H):
                bb = lax.broadcast(wv[hh], (16,))
                rows[e, pl.ds(hh * F, F)] = rows[e, pl.ds(hh * F, F)] * bb
            return carry2

        lax.fori_loop(0, CH, edge_body, 0)
        pltpu.sync_copy(rows, acc.at[idx_d], add=True)
        return carry

    lax.fori_loop(0, CHUNKS, chunk_body, 0)
    plsc.subcore_barrier()
    pltpu.sync_copy(acc.at[pl.ds(s * ROWS_PER_TILE, ROWS_PER_TILE)],
                    out_hbm.at[c, pl.ds(s * ROWS_PER_TILE, ROWS_PER_TILE)])


def _block_diag(a):
    eye = jnp.eye(H, dtype=jnp.float32)
    return (a[:, :, None] * eye[:, None, :]).reshape(D, H)


def kernel(n_feat, edge_index, e_feat, W1, al1, ar1, b1, W2, al2, ar2, b2):
    del e_feat
    ei = edge_index.astype(jnp.int32)
    pad_e = E_PAD - E_RAW
    src = jnp.concatenate([ei[0], jnp.zeros((pad_e,), jnp.int32)])
    dst = jnp.concatenate([ei[1], jnp.full((pad_e,), N, jnp.int32)])
    zero_blk = jnp.zeros((ROWS_PER_TILE, TW), jnp.float32)
    rep = jnp.repeat(jnp.eye(H, dtype=jnp.float32), F, axis=1)

    alr1 = jnp.concatenate([_block_diag(al1), _block_diag(ar1)], axis=1)
    arr1 = jnp.concatenate([_block_diag(ar1), _block_diag(ar1)], axis=1)
    alr2 = jnp.concatenate([_block_diag(al2), _block_diag(ar2)], axis=1)
    arr2 = jnp.concatenate([_block_diag(ar2), _block_diag(ar2)], axis=1)

    x = jnp.pad(n_feat, ((0, N_ACC - N), (0, 0)))
    t1, er1 = _embed(x, W1, alr1, arr1, rows_blk=2528)
    sc_edge = _make_sc_edge()
    acc1 = sc_edge(t1, er1, src, dst, zero_blk)
    t2, er2 = _norm_embed(acc1, rep, b1.reshape(1, D), W2, alr2, arr2,
                          rows_blk=2528)
    acc2 = sc_edge(t2, er2, src, dst, zero_blk)
    return _final(acc2, rep, b2.reshape(1, D), rows_blk=2000)
